# baseline (speedup 1.0000x reference)
import functools

import jax
import jax.numpy as jnp
from jax import lax
from jax.experimental import pallas as pl
from jax.experimental.pallas import tpu as pltpu

K_TOP = 100
K_LR = 0.1

_P1_CHUNKS = {1: 7, 2: 7, 3: 6, 4: 6, 5: 6}
_P2_CHUNKS = {6: 5, 7: 5}


def _ordered_key_u32(s):
    u = lax.bitcast_convert_type(s, jnp.uint32)
    flip = jnp.where(u >= jnp.uint32(0x80000000),
                     jnp.uint32(0xFFFFFFFF), jnp.uint32(0x80000000))
    return u ^ flip


def _p1_iter(key, p, bit):
    cand = p | jnp.uint32(1 << bit)
    cnt = jnp.sum((key >= cand[None, :]).astype(jnp.int32), axis=0)
    return jnp.where(cnt >= K_TOP, cand, p)


def _p2_iter(key, p, iota, need, lo, hi):
    eq = key == p[None, :]
    mid = (lo + hi) // 2
    c = jnp.sum((eq & (iota < mid[None, :])).astype(jnp.int32), axis=0)
    cond = c >= need
    return jnp.where(cond, mid, hi), jnp.where(cond, lo, mid)


def _full_search(key, iota):
    p = jnp.zeros((key.shape[1],), dtype=jnp.uint32)
    for bit in range(31, -1, -1):
        p = _p1_iter(key, p, bit)
    count_gt = jnp.sum((key > p[None, :]).astype(jnp.int32), axis=0)
    need = K_TOP - count_gt
    lo = jnp.zeros_like(count_gt)
    hi = jnp.full_like(count_gt, key.shape[0])
    for _ in range(10):
        hi, lo = _p2_iter(key, p, iota, need, lo, hi)
    return p, hi


def _fused_body(scores_ref, scoresnx_ref, oldloc_ref, oldscale_ref, at_ref,
                out_ref, thr_ref, idxthr_ref, pnx_ref, neednx_ref,
                lonx_ref, hinx_ref, acc_ref, accsq_ref):
    bi = pl.program_id(0)
    ni = pl.program_id(1)
    nn = pl.num_programs(1)
    nb = at_ref.shape[0]
    n = scores_ref.shape[0]
    bb = scores_ref.shape[1]

    @pl.when(ni == 0)
    def _start_block():
        @pl.when(bi == 0)
        def _cold():
            s = scores_ref[...]
            key = _ordered_key_u32(s)
            iota = lax.broadcasted_iota(jnp.int32, (n, bb), 0)
            p, hi = _full_search(key, iota)
            thr_ref[...] = p
            idxthr_ref[...] = hi

        @pl.when(bi > 0)
        def _handoff():
            thr_ref[...] = pnx_ref[...]
            idxthr_ref[...] = hinx_ref[...]

        pnx_ref[...] = jnp.zeros_like(pnx_ref)
        acc_ref[...] = jnp.zeros_like(acc_ref)
        accsq_ref[...] = jnp.zeros_like(accsq_ref)

    s = scores_ref[pl.ds(ni * nb, nb), :]
    key = _ordered_key_u32(s)
    thr = thr_ref[...][None, :]
    idxthr = idxthr_ref[...][None, :]
    iota = ni * nb + lax.broadcasted_iota(jnp.int32, (nb, bb), 0)
    m = (key > thr) | ((key == thr) & (iota < idxthr))
    mf = m.astype(jnp.float32)

    a = at_ref[...]
    am = a * mf[:, None, :]
    acc_ref[...] += jnp.sum(am, axis=0)
    accsq_ref[...] += jnp.sum(am * am, axis=0)

    bit_hi = 31
    for step, cnt in _P1_CHUNKS.items():
        @pl.when(ni == step)
        def _p1_chunk(bit_hi=bit_hi, cnt=cnt):
            keyn = _ordered_key_u32(scoresnx_ref[...])
            p = pnx_ref[...]
            for k in range(cnt):
                p = _p1_iter(keyn, p, bit_hi - k)
            pnx_ref[...] = p
        bit_hi -= cnt

    first_p2 = True
    for step, cnt in _P2_CHUNKS.items():
        @pl.when(ni == step)
        def _p2_chunk(cnt=cnt, first_p2=first_p2):
            keyn = _ordered_key_u32(scoresnx_ref[...])
            p = pnx_ref[...]
            iotan = lax.broadcasted_iota(jnp.int32, (n, bb), 0)
            if first_p2:
                count_gt = jnp.sum((keyn > p[None, :]).astype(jnp.int32),
                                   axis=0)
                need = K_TOP - count_gt
                lo = jnp.zeros_like(count_gt)
                hi = jnp.full_like(count_gt, n)
            else:
                need = neednx_ref[...]
                lo = lonx_ref[...]
                hi = hinx_ref[...]
            for _ in range(cnt):
                hi, lo = _p2_iter(keyn, p, iotan, need, lo, hi)
            neednx_ref[...] = need
            lonx_ref[...] = lo
            hinx_ref[...] = hi
        first_p2 = False

    @pl.when(ni == nn - 1)
    def _finalize():
        tot = acc_ref[...]
        totsq = accsq_ref[...]
        mean = tot * (1.0 / K_TOP)
        var = (totsq - tot * mean) * (1.0 / (K_TOP - 1))
        scale = jnp.sqrt(var + 1e-6)
        new_loc = (1.0 - K_LR) * oldloc_ref[...] + K_LR * mean
        new_scale = (1.0 - K_LR) * oldscale_ref[...] + K_LR * scale
        out_ref[...] = jnp.stack([new_loc, new_scale], axis=0)


@jax.jit
def kernel(obj, actions, old_loc, old_scale):
    N, B, V = actions.shape
    scores = obj[..., 0]
    at = jnp.moveaxis(actions, -1, 1)
    oldloc_t = old_loc.T
    oldscale_t = old_scale.T
    BB = 512
    NB = 128
    nbb = B // BB

    out_t = pl.pallas_call(
        _fused_body,
        grid=(nbb, N // NB),
        in_specs=[
            pl.BlockSpec((N, BB), lambda bi, ni: (0, bi)),
            pl.BlockSpec((N, BB),
                         lambda bi, ni: (0, jnp.minimum(bi + 1, nbb - 1))),
            pl.BlockSpec((V, BB), lambda bi, ni: (0, bi)),
            pl.BlockSpec((V, BB), lambda bi, ni: (0, bi)),
            pl.BlockSpec((NB, V, BB), lambda bi, ni: (ni, 0, bi)),
        ],
        out_specs=pl.BlockSpec((2, V, BB), lambda bi, ni: (0, 0, bi)),
        out_shape=jax.ShapeDtypeStruct((2, V, B), jnp.float32),
        scratch_shapes=[pltpu.VMEM((BB,), jnp.uint32),
                        pltpu.VMEM((BB,), jnp.int32),
                        pltpu.VMEM((BB,), jnp.uint32),
                        pltpu.VMEM((BB,), jnp.int32),
                        pltpu.VMEM((BB,), jnp.int32),
                        pltpu.VMEM((BB,), jnp.int32),
                        pltpu.VMEM((V, BB), jnp.float32),
                        pltpu.VMEM((V, BB), jnp.float32)],
    )(scores, scores, oldloc_t, oldscale_t, at)

    return jnp.moveaxis(out_t, 1, -1)
